# baseline (speedup 1.0000x reference)
import functools

import jax
import jax.numpy as jnp
from jax import lax
from jax.experimental import pallas as pl
from jax.experimental.pallas import tpu as pltpu
from jax.experimental.pallas import tpu_sc as plsc


def _make_kernel(B, F, V, D, NW):
    CB = 1024
    assert B % CB == 0
    nchunks = F * (B // CB)
    assert nchunks % NW == 0
    cpw = nchunks // NW
    nb = B // CB

    mesh = plsc.VectorSubcoreMesh(core_axis_name="c", subcore_axis_name="s")

    @functools.partial(
        pl.kernel,
        mesh=mesh,
        compiler_params=pltpu.CompilerParams(use_tc_tiling_on_sc=False),
        out_type=jax.ShapeDtypeStruct((F, B, D), jnp.float32),
        scratch_types=[
            pltpu.VMEM((2, CB), jnp.int32),
            pltpu.VMEM((2, CB, D), jnp.float32),
            pltpu.SemaphoreType.DMA,
            pltpu.SemaphoreType.DMA,
        ],
    )
    def emb(xt_hbm, table_hbm, out_hbm, idx_v, rows_v, sem0, sem1):
        wid = lax.axis_index("s") * 2 + lax.axis_index("c")
        cid0 = wid * cpw
        sems = (sem0, sem1)

        def fb(k):
            cid = cid0 + k
            return cid // nb, (cid % nb) * CB

        def start(k):
            s = k % 2
            f, b0 = fb(k)
            pltpu.sync_copy(xt_hbm.at[f, pl.ds(b0, CB)], idx_v.at[s])
            return pltpu.async_copy(
                table_hbm.at[f].at[idx_v.at[s]], rows_v.at[s], sems[s]
            )

        pending = start(0)
        for k in range(cpw):
            nxt = start(k + 1) if k + 1 < cpw else None
            pending.wait()
            f, b0 = fb(k)
            pltpu.sync_copy(rows_v.at[k % 2], out_hbm.at[f, pl.ds(b0, CB)])
            pending = nxt

    return emb


def kernel(x, W):
    B, F = x.shape
    _, V, D = W.shape
    info = plsc.get_sparse_core_info()
    NW = info.num_cores * info.num_subcores
    out = _make_kernel(B, F, V, D, NW)(x.T, W)
    return out.transpose(1, 0, 2).reshape(B, F * D)
